# baseline (speedup 1.0000x reference)
import functools

import jax
import jax.numpy as jnp
from jax import lax
from jax.experimental import pallas as pl
from jax.experimental.pallas import tpu as pltpu
from jax.experimental.pallas import tpu_sc as plsc

_VOCAB = 100000
_D = 64
_BATCH = 16384
_HIST = 50
_TOTAL = _BATCH * _HIST

_NC = 2
_NS = 16
_NW = _NC * _NS
_NBLK = _HIST * (_BATCH // 128)
_PER_W = _NBLK // _NW
_CHUNK = 128
_NBUF = 5
_BLKW = _CHUNK * _D


def _emb_body(table, xr, out, idx_v, rows_v, tbuf, *sems):
    gsems = sems[:_NBUF]
    ssems = sems[_NBUF:]
    c = lax.axis_index("c")
    s = lax.axis_index("s")
    wid = s * _NC + c
    base = wid * _PER_W

    pltpu.sync_copy(xr.at[wid], idx_v)

    lane = lax.iota(jnp.int32, 16)
    rowvecs = [
        [b * _D + k * 16 + lane for k in range(4)] for b in range(_NBUF)
    ]

    def fire_gather(j, b):
        return pltpu.async_copy(table.at[idx_v.at[j]], rows_v.at[b], gsems[b])

    def transpose(b):
        @plsc.parallel_loop(0, _CHUNK, unroll=2)
        def tcol(b2):
            col = jnp.full((16,), b2, jnp.int32)
            for k in range(4):
                v = rows_v[b, b2, pl.ds(k * 16, 16)]
                plsc.store_scatter(tbuf, [rowvecs[b][k], col], v)

    def fire_stores(j, b):
        g = base + j
        h = g // 128
        bt = lax.rem(g, 128)
        obase = (h * 8 * 128 + bt) * 8
        hs = []
        for dt in range(8):
            hs.append(
                pltpu.async_copy(
                    tbuf.at[pl.ds(b * _D + dt * 8, 8), pl.ds(0, 128)],
                    out.at[pl.ds(obase + dt * 128 * 8, 8)],
                    ssems[b],
                )
            )
        return hs

    def wait_stores(b):
        for dt in range(8):
            pltpu.make_async_copy(
                tbuf.at[pl.ds(b * _D + dt * 8, 8), pl.ds(0, 128)],
                out.at[pl.ds(0, 8)],
                ssems[b],
            ).wait()

    gh = [fire_gather(b, b) for b in range(_NBUF)]
    for b in range(_NBUF):
        gh[b].wait()
        transpose(b)
        fire_stores(b, b)

    def group(it, carry):
        j0 = it * _NBUF
        gh = []
        for b in range(_NBUF):
            wait_stores(b)
            gh.append(fire_gather(j0 + b, b))
        for b in range(_NBUF):
            gh[b].wait()
            transpose(b)
            fire_stores(j0 + b, b)
        return carry

    lax.fori_loop(1, _PER_W // _NBUF, group, 0)

    for b in range(_NBUF):
        wait_stores(b)


@jax.jit
def _emb(x, embed_weight):
    xr = x.T.reshape(_NW, _PER_W, _CHUNK)
    mesh = plsc.VectorSubcoreMesh(core_axis_name="c", subcore_axis_name="s")
    scratch = [
        pltpu.VMEM((_PER_W, _CHUNK), jnp.int32),
        pltpu.VMEM((_NBUF, _CHUNK, _D), jnp.float32),
        pltpu.VMEM((_NBUF * _D, 129), jnp.float32),
    ] + [pltpu.SemaphoreType.DMA] * (2 * _NBUF)
    out1 = pl.kernel(
        _emb_body,
        out_type=jax.ShapeDtypeStruct((_HIST * 8 * 128 * 8, 128), jnp.float32),
        mesh=mesh,
        scratch_types=scratch,
        compiler_params=pltpu.CompilerParams(
            use_tc_tiling_on_sc=False, needs_layout_passes=False
        ),
    )(embed_weight, xr)
    out5d = out1.reshape(_HIST, 8, 128, 8, 128)
    return out5d.transpose(2, 4, 0, 1, 3).reshape(_BATCH, _HIST, _D)


def kernel(x, embed_weight):
    return _emb(x, embed_weight)


# speedup vs baseline: 1.0008x; 1.0008x over previous
import functools

import jax
import jax.numpy as jnp
from jax import lax
from jax.experimental import pallas as pl
from jax.experimental.pallas import tpu as pltpu
from jax.experimental.pallas import tpu_sc as plsc

_VOCAB = 100000
_D = 64
_BATCH = 16384
_HIST = 50
_TOTAL = _BATCH * _HIST

_NC = 2
_NS = 16
_NW = _NC * _NS
_NBLK = _HIST * (_BATCH // 128)
_PER_W = _NBLK // _NW
_CHUNK = 128
_NBUF = 5
_BLKW = _CHUNK * _D


def _emb_body(table, xr, out, idx_v, rows_v, tbuf, *sems):
    gsems = sems[:_NBUF]
    ssems = sems[_NBUF:]
    c = lax.axis_index("c")
    s = lax.axis_index("s")
    wid = s * _NC + c
    base = wid * _PER_W

    pltpu.sync_copy(xr.at[wid], idx_v)

    lane = lax.iota(jnp.int32, 16)
    rowvecs = [
        [b * _D + k * 16 + lane for k in range(4)] for b in range(_NBUF)
    ]

    def fire_gather(j, b):
        return pltpu.async_copy(table.at[idx_v.at[j]], rows_v.at[b], gsems[b])

    def transpose(b):
        @plsc.parallel_loop(0, _CHUNK, unroll=4)
        def tcol(b2):
            col = jnp.full((16,), b2, jnp.int32)
            for k in range(4):
                v = rows_v[b, b2, pl.ds(k * 16, 16)]
                plsc.store_scatter(tbuf, [rowvecs[b][k], col], v)

    def fire_stores(j, b):
        g = base + j
        h = g // 128
        bt = lax.rem(g, 128)
        obase = (h * 8 * 128 + bt) * 8
        hs = []
        for dt in range(8):
            hs.append(
                pltpu.async_copy(
                    tbuf.at[pl.ds(b * _D + dt * 8, 8), pl.ds(0, 128)],
                    out.at[pl.ds(obase + dt * 128 * 8, 8)],
                    ssems[b],
                )
            )
        return hs

    def wait_stores(b):
        for dt in range(8):
            pltpu.make_async_copy(
                tbuf.at[pl.ds(b * _D + dt * 8, 8), pl.ds(0, 128)],
                out.at[pl.ds(0, 8)],
                ssems[b],
            ).wait()

    gh = [fire_gather(b, b) for b in range(_NBUF)]
    for b in range(_NBUF):
        gh[b].wait()
        transpose(b)
        fire_stores(b, b)

    def group(it, carry):
        j0 = it * _NBUF
        gh = []
        for b in range(_NBUF):
            wait_stores(b)
            gh.append(fire_gather(j0 + b, b))
        for b in range(_NBUF):
            gh[b].wait()
            transpose(b)
            fire_stores(j0 + b, b)
        return carry

    lax.fori_loop(1, _PER_W // _NBUF, group, 0)

    for b in range(_NBUF):
        wait_stores(b)


@jax.jit
def _emb(x, embed_weight):
    xr = x.T.reshape(_NW, _PER_W, _CHUNK)
    mesh = plsc.VectorSubcoreMesh(core_axis_name="c", subcore_axis_name="s")
    scratch = [
        pltpu.VMEM((_PER_W, _CHUNK), jnp.int32),
        pltpu.VMEM((_NBUF, _CHUNK, _D), jnp.float32),
        pltpu.VMEM((_NBUF * _D, 129), jnp.float32),
    ] + [pltpu.SemaphoreType.DMA] * (2 * _NBUF)
    out1 = pl.kernel(
        _emb_body,
        out_type=jax.ShapeDtypeStruct((_HIST * 8 * 128 * 8, 128), jnp.float32),
        mesh=mesh,
        scratch_types=scratch,
        compiler_params=pltpu.CompilerParams(
            use_tc_tiling_on_sc=False, needs_layout_passes=False
        ),
    )(embed_weight, xr)
    out5d = out1.reshape(_HIST, 8, 128, 8, 128)
    return out5d.transpose(2, 4, 0, 1, 3).reshape(_BATCH, _HIST, _D)


def kernel(x, embed_weight):
    return _emb(x, embed_weight)
